# baseline (speedup 1.0000x reference)
import functools
import jax
import jax.numpy as jnp
from jax import lax
from jax.experimental import pallas as pl
from jax.experimental.pallas import tpu as pltpu
from jax.experimental.pallas import tpu_sc as plsc

B, C, Hh, Ww = 8, 384, 32, 32
D = 64
K = 1024
HP = Hh + 2
ROWS = B * HP * HP
PAD0 = 48
EXT = 9344
IEXT = 9472
VQC = 8
VQR = EXT // VQC
CC = 4
CR = ROWS // CC
SHIFTS = [(dh - 1) * HP + (dw - 1) for dh in range(3) for dw in range(3)]
GROUPS = [(0, 0, 4), (1, 4, 4), (2, 8, 1)]
f32 = jnp.float32
bf16 = jnp.bfloat16


def _conv9(x_ref, w_ref, b_row, mask_ref, relu, nout, base):
    parts = []
    for gi, g0, gn in GROUPS:
        s0 = SHIFTS[g0]
        span = CR + (SHIFTS[g0 + gn - 1] - s0)
        x = x_ref[base + s0:base + s0 + span, :]
        if x.dtype != bf16:
            x = x.astype(bf16)
        y = jax.lax.dot_general(x, w_ref[gi], (((1,), (0,)), ((), ())),
                                preferred_element_type=f32)
        for i in range(gn):
            d = SHIFTS[g0 + i] - s0
            parts.append(y[d:d + CR, i * nout:(i + 1) * nout])
    acc = None
    for p in parts:
        acc = p if acc is None else acc + p
    acc = acc + b_row
    if relu:
        acc = jnp.maximum(acc, 0.0)
    return acc * mask_ref[base:base + CR, :]


def _enc_kernel(h_ref, wq1_ref, bq1_ref, wq2_ref, bq2_ref, cb_ref, mask_ref,
                ze_ref, idx_ref, z1_ref):
    z1_ref[...] = jnp.zeros((EXT, D), bf16)
    ze_ref[...] = jnp.zeros((EXT, D), f32)
    for c in range(CC):
        base = PAD0 + c * CR
        z1 = _conv9(h_ref, wq1_ref, bq1_ref[0:1, :], mask_ref, True, D, base)
        z1_ref[base:base + CR, :] = z1.astype(bf16)
    for c in range(CC):
        base = PAD0 + c * CR
        ze = _conv9(z1_ref, wq2_ref, bq2_ref[0:1, :], mask_ref, False, D, base)
        ze_ref[base:base + CR, :] = ze

    cb = cb_ref[...]
    cb_b = cb.astype(bf16)
    cnorm = jnp.sum(cb * cb, axis=1, keepdims=True).reshape(1, K)
    for c in range(VQC):
        z = ze_ref[c * VQR:(c + 1) * VQR, :]
        m = jax.lax.dot_general(z.astype(bf16), cb_b, (((1,), (1,)), ((), ())),
                                preferred_element_type=f32)
        znorm = jnp.sum(z * z, axis=1, keepdims=True)
        dist = (znorm - 2.0 * m) + cnorm
        minv = jnp.min(dist, axis=1, keepdims=True)
        iot = jax.lax.broadcasted_iota(jnp.int32, (VQR, K), 1)
        idx = jnp.min(jnp.where(dist == minv, iot, K), axis=1, keepdims=True)
        idx_ref[c * VQR:(c + 1) * VQR, :] = idx
    idx_ref[EXT:IEXT, :] = jnp.zeros((IEXT - EXT, 1), jnp.int32)


_SC_INFO = plsc.get_sparse_core_info()
_NW = _SC_INFO.num_cores * _SC_INFO.num_subcores
_BPW = IEXT // _NW


@functools.partial(
    pl.kernel,
    mesh=plsc.VectorSubcoreMesh(core_axis_name="c", subcore_axis_name="s"),
    out_type=jax.ShapeDtypeStruct((IEXT, 128), f32),
    scratch_types=[
        pltpu.VMEM((_BPW,), jnp.int32),
        pltpu.VMEM((_BPW, 128), f32),
        pltpu.SemaphoreType.DMA,
    ],
)
def _sc_gather(table_hbm, idx_hbm, out_hbm, idx_v, rows_v, sem):
    wid = lax.axis_index("s") * _SC_INFO.num_cores + lax.axis_index("c")
    base = wid * _BPW
    pltpu.sync_copy(idx_hbm.at[pl.ds(base, _BPW)], idx_v)
    pltpu.async_copy(table_hbm.at[idx_v], rows_v, sem).wait()
    pltpu.sync_copy(rows_v, out_hbm.at[pl.ds(base, _BPW)])


def _dec_kernel(zq_ref, ze_ref, wr1_ref, br1_ref, wr2_ref, br2_ref, wsk_ref,
                bsk_ref, mask_ref, hhat_ref, loss_ref, zqm_ref, r1_ref):
    zqm_f = zq_ref[0:EXT, 0:D] * mask_ref[...]
    zqm_ref[...] = zqm_f.astype(bf16)
    diff = ze_ref[...] - zqm_f
    loss_ref[...] = jnp.sum(diff * diff).reshape(1, 1) * (1.0 / (B * Hh * Ww * D))

    r1_ref[...] = jnp.zeros((EXT, C), bf16)
    for c in range(CC):
        base = PAD0 + c * CR
        r1 = _conv9(zqm_ref, wr1_ref, br1_ref[0:1, :], mask_ref, True, C, base)
        r1_ref[base:base + CR, :] = r1.astype(bf16)
    for c in range(CC):
        base = PAD0 + c * CR
        parts = []
        for gi, g0, gn in GROUPS:
            s0 = SHIFTS[g0]
            span = CR + (SHIFTS[g0 + gn - 1] - s0)
            x = r1_ref[base + s0:base + s0 + span, :]
            y = jax.lax.dot_general(x, wr2_ref[gi], (((1,), (0,)), ((), ())),
                                    preferred_element_type=f32)
            for i in range(gn):
                d = SHIFTS[g0 + i] - s0
                parts.append(y[d:d + CR, i * C:(i + 1) * C])
        acc = None
        for p in parts:
            acc = p if acc is None else acc + p
        ysk = jax.lax.dot_general(zqm_ref[base:base + CR, :], wsk_ref[...],
                                  (((1,), (0,)), ((), ())),
                                  preferred_element_type=f32)
        hhat_ref[(base - PAD0):(base - PAD0) + CR, :] = (
            (acc + br2_ref[0:1, :]) + (ysk + bsk_ref[0:1, :]))


def _packw(wt, nout):
    slabs = []
    for gi, g0, gn in GROUPS:
        cat = jnp.concatenate([wt[g0 + i] for i in range(gn)], axis=1)
        if gn < 4:
            cat = jnp.pad(cat, ((0, 0), (0, (4 - gn) * nout)))
        slabs.append(cat)
    return jnp.stack(slabs)


def kernel(h, Wq1, bq1, Wq2, bq2, codebook, Wr1, br1, Wr2, br2, Wskip, bskip):
    hp = jnp.pad(jnp.transpose(h, (0, 2, 3, 1)), ((0, 0), (1, 1), (1, 1), (0, 0)))
    hflat = hp.reshape(ROWS, C)
    h_ext = jnp.pad(hflat, ((PAD0, EXT - PAD0 - ROWS), (0, 0))).astype(bf16)

    wq1 = _packw(jnp.transpose(Wq1, (2, 3, 1, 0)).reshape(9, C, D).astype(bf16), D)
    wq2 = _packw(jnp.transpose(Wq2, (2, 3, 1, 0)).reshape(9, D, D).astype(bf16), D)
    wr1 = _packw(jnp.transpose(Wr1, (2, 3, 1, 0)).reshape(9, D, C).astype(bf16), C)
    wr2 = _packw(jnp.transpose(Wr2, (2, 3, 1, 0)).reshape(9, C, C).astype(bf16), C)
    wsk = jnp.transpose(Wskip, (2, 3, 1, 0)).reshape(D, C).astype(bf16)

    r = jnp.arange(EXT) - PAD0
    j = r % (HP * HP) % HP
    i = r % (HP * HP) // HP
    valid = (r >= 0) & (r < ROWS) & (i >= 1) & (i <= Hh) & (j >= 1) & (j <= Ww)
    mask = valid.astype(f32)[:, None]

    ze_ext, idx_ext = pl.pallas_call(
        _enc_kernel,
        out_shape=(
            jax.ShapeDtypeStruct((EXT, D), f32),
            jax.ShapeDtypeStruct((IEXT, 1), jnp.int32),
        ),
        scratch_shapes=[pltpu.VMEM((EXT, D), bf16)],
        compiler_params=pltpu.CompilerParams(
            vmem_limit_bytes=100 * 1024 * 1024,
        ),
    )(h_ext, wq1, bq1.reshape(1, D), wq2, bq2.reshape(1, D), codebook, mask)

    zq_raw = _sc_gather(jnp.pad(codebook, ((0, 0), (0, 128 - D))),
                        idx_ext.reshape(IEXT))

    hhat_rows, loss = pl.pallas_call(
        _dec_kernel,
        out_shape=(
            jax.ShapeDtypeStruct((ROWS, C), f32),
            jax.ShapeDtypeStruct((1, 1), f32),
        ),
        scratch_shapes=[
            pltpu.VMEM((EXT, D), bf16),
            pltpu.VMEM((EXT, C), bf16),
        ],
        compiler_params=pltpu.CompilerParams(
            vmem_limit_bytes=100 * 1024 * 1024,
        ),
    )(zq_raw, ze_ext, wr1, br1.reshape(1, C), wr2, br2.reshape(1, C), wsk,
      bskip.reshape(1, C), mask)

    zq = zq_raw[PAD0:PAD0 + ROWS, 0:D].reshape(B, HP, HP, D)[:, 1:1 + Hh, 1:1 + Ww, :]
    z_q_st = jnp.transpose(zq, (0, 3, 1, 2))
    hh = hhat_rows.reshape(B, HP, HP, C)[:, 1:1 + Hh, 1:1 + Ww, :]
    h_hat = jnp.transpose(hh, (0, 3, 1, 2))
    indices = idx_ext[PAD0:PAD0 + ROWS, 0].reshape(B, HP, HP)[:, 1:1 + Hh, 1:1 + Ww]
    return (z_q_st, h_hat, loss.reshape(()), indices)


# speedup vs baseline: 1.3446x; 1.3446x over previous
import jax
import jax.numpy as jnp
from jax.experimental import pallas as pl
from jax.experimental.pallas import tpu as pltpu

B, C, Hh, Ww = 8, 384, 32, 32
D = 64
K = 1024
HP = Hh + 2
ROWS = B * HP * HP
PAD0 = 48
EXT = 9344
VQC = 4
VQR = EXT // VQC
CC = 4
CR = ROWS // CC
SHIFTS = [(dh - 1) * HP + (dw - 1) for dh in range(3) for dw in range(3)]
GROUPS = [(0, 0, 4), (1, 4, 4), (2, 8, 1)]
f32 = jnp.float32
bf16 = jnp.bfloat16


def _conv9(x_ref, w_ref, b_row, mask_ref, relu, nout, base):
    parts = []
    for gi, g0, gn in GROUPS:
        s0 = SHIFTS[g0]
        span = CR + (SHIFTS[g0 + gn - 1] - s0)
        x = x_ref[base + s0:base + s0 + span, :]
        if x.dtype != bf16:
            x = x.astype(bf16)
        y = jax.lax.dot_general(x, w_ref[gi], (((1,), (0,)), ((), ())),
                                preferred_element_type=f32)
        for i in range(gn):
            d = SHIFTS[g0 + i] - s0
            parts.append(y[d:d + CR, i * nout:(i + 1) * nout])
    acc = None
    for p in parts:
        acc = p if acc is None else acc + p
    acc = acc + b_row
    if relu:
        acc = jnp.maximum(acc, 0.0)
    return acc * mask_ref[base:base + CR, :]


def _enc_kernel(h_ref, wq1_ref, bq1_ref, wq2_ref, bq2_ref, cb_ref, mask_ref,
                zq_ref, idx_ref, loss_ref, z1_ref, ze_ref):
    z1_ref[...] = jnp.zeros((EXT, D), bf16)
    ze_ref[...] = jnp.zeros((EXT, D), f32)
    for c in range(CC):
        base = PAD0 + c * CR
        z1 = _conv9(h_ref, wq1_ref, bq1_ref[0:1, :], mask_ref, True, D, base)
        z1_ref[base:base + CR, :] = z1.astype(bf16)
    for c in range(CC):
        base = PAD0 + c * CR
        ze = _conv9(z1_ref, wq2_ref, bq2_ref[0:1, :], mask_ref, False, D, base)
        ze_ref[base:base + CR, :] = ze

    cb = cb_ref[...]
    cb_b = cb.astype(bf16)
    cb_lo = (cb - cb_b.astype(f32)).astype(bf16)
    cnorm = jnp.sum(cb * cb, axis=1, keepdims=True).reshape(1, K)
    acc_loss = jnp.zeros((1, 1), f32)
    for c in range(VQC):
        z = ze_ref[c * VQR:(c + 1) * VQR, :]
        m = jax.lax.dot_general(z.astype(bf16), cb_b, (((1,), (1,)), ((), ())),
                                preferred_element_type=f32)
        znorm = jnp.sum(z * z, axis=1, keepdims=True)
        dist = (znorm - 2.0 * m) + cnorm
        minv = jnp.min(dist, axis=1, keepdims=True)
        iot = jax.lax.broadcasted_iota(jnp.int32, (VQR, K), 1)
        idx = jnp.min(jnp.where(dist == minv, iot, K), axis=1, keepdims=True)
        idx_ref[c * VQR:(c + 1) * VQR, :] = idx
        onehot = (iot == idx).astype(bf16)
        zq = (jax.lax.dot_general(onehot, cb_b, (((1,), (0,)), ((), ())),
                                  preferred_element_type=f32)
              + jax.lax.dot_general(onehot, cb_lo, (((1,), (0,)), ((), ())),
                                    preferred_element_type=f32))
        zq = zq * mask_ref[c * VQR:(c + 1) * VQR, :]
        zq_ref[c * VQR:(c + 1) * VQR, :] = zq
        diff = z - zq
        acc_loss = acc_loss + jnp.sum(diff * diff).reshape(1, 1)
    loss_ref[...] = acc_loss * (1.0 / (B * Hh * Ww * D))


def _dec_kernel(zq_ref, wr1_ref, br1_ref, wr2_ref, br2_ref, wsk_ref, bsk_ref,
                mask_ref, hhat_ref, r1_ref):
    r1_ref[...] = jnp.zeros((EXT, C), bf16)
    for c in range(CC):
        base = PAD0 + c * CR
        r1 = _conv9(zq_ref, wr1_ref, br1_ref[0:1, :], mask_ref, True, C, base)
        r1_ref[base:base + CR, :] = r1.astype(bf16)
    for c in range(CC):
        base = PAD0 + c * CR
        parts = []
        for gi, g0, gn in GROUPS:
            s0 = SHIFTS[g0]
            span = CR + (SHIFTS[g0 + gn - 1] - s0)
            x = r1_ref[base + s0:base + s0 + span, :]
            y = jax.lax.dot_general(x, wr2_ref[gi], (((1,), (0,)), ((), ())),
                                    preferred_element_type=f32)
            for i in range(gn):
                d = SHIFTS[g0 + i] - s0
                parts.append(y[d:d + CR, i * C:(i + 1) * C])
        acc = None
        for p in parts:
            acc = p if acc is None else acc + p
        ysk = jax.lax.dot_general(zq_ref[base:base + CR, :].astype(bf16),
                                  wsk_ref[...], (((1,), (0,)), ((), ())),
                                  preferred_element_type=f32)
        hhat_ref[(base - PAD0):(base - PAD0) + CR, :] = (
            (acc + br2_ref[0:1, :]) + (ysk + bsk_ref[0:1, :]))


def _packw(wt, nout):
    slabs = []
    for gi, g0, gn in GROUPS:
        cat = jnp.concatenate([wt[g0 + i] for i in range(gn)], axis=1)
        if gn < 4:
            cat = jnp.pad(cat, ((0, 0), (0, (4 - gn) * nout)))
        slabs.append(cat)
    return jnp.stack(slabs)


def kernel(h, Wq1, bq1, Wq2, bq2, codebook, Wr1, br1, Wr2, br2, Wskip, bskip):
    hp = jnp.pad(jnp.transpose(h, (0, 2, 3, 1)), ((0, 0), (1, 1), (1, 1), (0, 0)))
    hflat = hp.reshape(ROWS, C)
    h_ext = jnp.pad(hflat, ((PAD0, EXT - PAD0 - ROWS), (0, 0))).astype(bf16)

    wq1 = _packw(jnp.transpose(Wq1, (2, 3, 1, 0)).reshape(9, C, D).astype(bf16), D)
    wq2 = _packw(jnp.transpose(Wq2, (2, 3, 1, 0)).reshape(9, D, D).astype(bf16), D)
    wr1 = _packw(jnp.transpose(Wr1, (2, 3, 1, 0)).reshape(9, D, C).astype(bf16), C)
    wr2 = _packw(jnp.transpose(Wr2, (2, 3, 1, 0)).reshape(9, C, C).astype(bf16), C)
    wsk = jnp.transpose(Wskip, (2, 3, 1, 0)).reshape(D, C).astype(bf16)

    r = jnp.arange(EXT) - PAD0
    j = r % (HP * HP) % HP
    i = r % (HP * HP) // HP
    valid = (r >= 0) & (r < ROWS) & (i >= 1) & (i <= Hh) & (j >= 1) & (j <= Ww)
    mask = valid.astype(f32)[:, None]

    zq_ext, idx_ext, loss = pl.pallas_call(
        _enc_kernel,
        out_shape=(
            jax.ShapeDtypeStruct((EXT, D), f32),
            jax.ShapeDtypeStruct((EXT, 1), jnp.int32),
            jax.ShapeDtypeStruct((1, 1), f32),
        ),
        scratch_shapes=[
            pltpu.VMEM((EXT, D), bf16),
            pltpu.VMEM((EXT, D), f32),
        ],
        compiler_params=pltpu.CompilerParams(
            vmem_limit_bytes=100 * 1024 * 1024,
        ),
    )(h_ext, wq1, bq1.reshape(1, D), wq2, bq2.reshape(1, D), codebook, mask)

    hhat_rows = pl.pallas_call(
        _dec_kernel,
        out_shape=jax.ShapeDtypeStruct((ROWS, C), f32),
        scratch_shapes=[pltpu.VMEM((EXT, C), bf16)],
        compiler_params=pltpu.CompilerParams(
            vmem_limit_bytes=100 * 1024 * 1024,
        ),
    )(zq_ext, wr1, br1.reshape(1, C), wr2, br2.reshape(1, C), wsk,
      bskip.reshape(1, C), mask)

    zq = zq_ext[PAD0:PAD0 + ROWS].reshape(B, HP, HP, D)[:, 1:1 + Hh, 1:1 + Ww, :]
    z_q_st = jnp.transpose(zq, (0, 3, 1, 2))
    hh = hhat_rows.reshape(B, HP, HP, C)[:, 1:1 + Hh, 1:1 + Ww, :]
    h_hat = jnp.transpose(hh, (0, 3, 1, 2))
    indices = idx_ext[PAD0:PAD0 + ROWS, 0].reshape(B, HP, HP)[:, 1:1 + Hh, 1:1 + Ww]
    return (z_q_st, h_hat, loss.reshape(()), indices)
